# baseline (speedup 1.0000x reference)
import functools

import jax
import jax.numpy as jnp
from jax import lax
from jax.experimental import pallas as pl
from jax.experimental.pallas import tpu as pltpu
from jax.experimental.pallas import tpu_sc as plsc

B, S, F = 4, 8192, 2048
NC, NS, L = 2, 16, 16
NW = NC * NS
WPB = NW // B
FPW = F // WPB
NLANES = FPW // L
R = 64
NCH = S // R
NBUF = 4
PD = NBUF - 1
NT = NCH // NBUF
assert NCH % NBUF == 0


def _sc_cumsum(x):
    mesh = plsc.VectorSubcoreMesh(core_axis_name="c", subcore_axis_name="s")

    @functools.partial(
        pl.kernel,
        mesh=mesh,
        out_type=jax.ShapeDtypeStruct((B, S, F), jnp.float32),
        scratch_types=[
            pltpu.VMEM((R, FPW), jnp.float32),
            pltpu.VMEM((R, FPW), jnp.float32),
            pltpu.VMEM((R, FPW), jnp.float32),
            pltpu.VMEM((R, FPW), jnp.float32),
            pltpu.SemaphoreType.DMA,
            pltpu.SemaphoreType.DMA,
        ],
    )
    def k(x_hbm, out_hbm, buf0, buf1, buf2, buf3, sem_in, sem_out):
        bufs = (buf0, buf1, buf2, buf3)
        wid = lax.axis_index("s") * NC + lax.axis_index("c")
        b = wid // WPB
        f0 = (wid % WPB) * FPW

        def src(kk):
            return x_hbm.at[b, pl.ds(kk * R, R), pl.ds(f0, FPW)]

        def dst(kk):
            return out_hbm.at[b, pl.ds(kk * R, R), pl.ds(f0, FPW)]

        def start_in(kk, buf):
            pltpu.make_async_copy(src(kk), buf, sem_in).start()

        def wait_in(buf):
            pltpu.make_async_copy(src(0), buf, sem_in).wait()

        def start_out(kk, buf):
            pltpu.make_async_copy(buf, dst(kk), sem_out).start()

        def wait_out(buf):
            pltpu.make_async_copy(buf, dst(0), sem_out).wait()

        def compute(buf, runs):
            def row_body(r, rs):
                new = []
                for c in range(NLANES):
                    sl = pl.ds(c * L, L)
                    v = rs[c] + buf[r, sl]
                    buf[r, sl] = v
                    new.append(v)
                return tuple(new)

            return lax.fori_loop(0, R, row_body, runs, unroll=4)

        runs0 = tuple(jnp.zeros((L,), jnp.float32) for _ in range(NLANES))
        for j in range(PD):
            start_in(j, bufs[j])

        def ring(t, runs):
            k0 = NBUF * t
            for j in range(NBUF):
                kk = k0 + j
                buf = bufs[j]
                wait_in(buf)
                runs = compute(buf, runs)
                start_out(kk, buf)
                nxt = bufs[(j + PD) % NBUF]

                @pl.when(kk + PD < NCH)
                def _():
                    @pl.when(kk >= 1)
                    def _():
                        wait_out(nxt)

                    start_in(kk + PD, nxt)

            return runs

        lax.fori_loop(0, NT, ring, runs0)
        for j in range(NBUF):
            wait_out(bufs[(j + 1) % NBUF])

    return k(x)


def kernel(x, dim, dtype):
    return _sc_cumsum(x)


# speedup vs baseline: 2.2507x; 2.2507x over previous
import jax
import jax.numpy as jnp
from jax.experimental import pallas as pl
from jax.experimental.pallas import tpu as pltpu

SEQ_BLK = 1024
FEAT_BLK = 2048
GROUP = 8


def _group_scan(v):
    s = v.shape[0]
    shift = 1
    while shift < s:
        pad = jnp.zeros((shift, v.shape[1]), v.dtype)
        v = v + jnp.concatenate([pad, v[:-shift]], axis=0)
        shift *= 2
    return v


def _body(x_ref, o_ref, carry_ref):
    sb = pl.program_id(2)

    @pl.when(sb == 0)
    def _():
        carry_ref[...] = jnp.zeros_like(carry_ref)

    def step(g, carry):
        v = x_ref[0, pl.ds(g * GROUP, GROUP), :]
        v = _group_scan(v) + carry
        o_ref[0, pl.ds(g * GROUP, GROUP), :] = v
        return v[GROUP - 1:GROUP, :]

    carry = jax.lax.fori_loop(0, SEQ_BLK // GROUP, step, carry_ref[...],
                              unroll=4)
    carry_ref[...] = carry


def kernel(x, dim, dtype):
    b, s, f = x.shape
    grid = (b, f // FEAT_BLK, s // SEQ_BLK)
    out = pl.pallas_call(
        _body,
        grid=grid,
        in_specs=[pl.BlockSpec((1, SEQ_BLK, FEAT_BLK),
                               lambda b_, f_, s_: (b_, s_, f_))],
        out_specs=pl.BlockSpec((1, SEQ_BLK, FEAT_BLK),
                               lambda b_, f_, s_: (b_, s_, f_)),
        out_shape=jax.ShapeDtypeStruct(x.shape, x.dtype),
        scratch_shapes=[pltpu.VMEM((1, FEAT_BLK), x.dtype)],
        compiler_params=pltpu.CompilerParams(
            dimension_semantics=("parallel", "parallel", "arbitrary"),
        ),
    )(x)
    return out
